# Initial kernel scaffold; baseline (speedup 1.0000x reference)
import jax
import jax.numpy as jnp
from jax.experimental import pallas as pl


def kernel(adj, node_emb, W1, b1, W2, b2, Wres, bres, Wfc1, bfc1, Wfc2, bfc2):
    raise NotImplementedError("write your pallas kernel here")


# speedup vs baseline: 3179.5856x; 3179.5856x over previous
import jax
import jax.numpy as jnp
from jax.experimental import pallas as pl

_N = 1024
_F32 = jnp.float32


def _fused(adj_ref, x_ref, W1_ref, b1_ref, W2_ref, b2_ref, Wres_ref,
           bres_ref, Wfc1_ref, bfc1_ref, Wfc2_ref, bfc2_ref,
           x_out_ref, A_out_ref):
    adj = adj_ref[...]
    x0 = x_ref[...]

    ones = jnp.ones((_N, 1), _F32)
    deg = jax.lax.dot_general(adj, ones, (((0,), (0,)), ((), ())),
                              preferred_element_type=_F32) + 2.0
    dinv = jax.lax.rsqrt(deg)
    dinv2 = 2.0 * dinv * dinv

    def conv(x, W_ref, b_ref):
        xw = jnp.dot(x, W_ref[...], preferred_element_type=_F32)
        t = jax.lax.dot_general(adj, dinv * xw, (((0,), (0,)), ((), ())),
                                preferred_element_type=_F32)
        return dinv * t + dinv2 * xw + b_ref[...]

    x1 = jax.nn.relu(conv(x0, W1_ref, b1_ref))
    x2 = jax.nn.relu(conv(x1, W2_ref, b2_ref))
    x = x2 + jnp.dot(x1, Wres_ref[...], preferred_element_type=_F32) \
        + bres_ref[...]
    h = jax.nn.relu(jnp.dot(x, Wfc1_ref[...], preferred_element_type=_F32)
                    + bfc1_ref[...])
    A = jnp.dot(h, Wfc2_ref[...], preferred_element_type=_F32) + bfc2_ref[...]

    x_out_ref[...] = x
    A_out_ref[...] = A


def kernel(adj, node_emb, W1, b1, W2, b2, Wres, bres, Wfc1, bfc1, Wfc2, bfc2):
    out = pl.pallas_call(
        _fused,
        out_shape=(
            jax.ShapeDtypeStruct((_N, 128), _F32),
            jax.ShapeDtypeStruct((_N, 1), _F32),
        ),
    )(adj, node_emb,
      W1, b1.reshape(1, -1), W2, b2.reshape(1, -1),
      Wres, bres.reshape(1, -1), Wfc1, bfc1.reshape(1, -1),
      Wfc2, bfc2.reshape(1, -1))
    return out


# speedup vs baseline: 3346.2560x; 1.0524x over previous
import jax
import jax.numpy as jnp
from jax.experimental import pallas as pl

_N = 1024
_F32 = jnp.float32


def _fused(adj_ref, x_ref, W1_ref, b1_ref, W2_ref, b2_ref, Wres_ref,
           bres_ref, Wfc1_ref, bfc1_ref, Wfc2_ref, bfc2_ref,
           x_out_ref, A_out_ref):
    adj = adj_ref[...]
    x0 = x_ref[...]

    adjb = adj.astype(jnp.bfloat16)

    deg = jnp.transpose(jnp.sum(adj, axis=0, keepdims=True)) + 2.0
    dinv = jax.lax.rsqrt(deg)
    dinv2 = 2.0 * dinv * dinv

    def conv(x, W_ref, b_ref):
        xw = jnp.dot(x.astype(jnp.bfloat16), W_ref[...].astype(jnp.bfloat16),
                     preferred_element_type=_F32)
        t = jax.lax.dot_general(adjb, (dinv * xw).astype(jnp.bfloat16),
                                (((0,), (0,)), ((), ())),
                                preferred_element_type=_F32)
        return dinv * t + dinv2 * xw + b_ref[...]

    x1 = jax.nn.relu(conv(x0, W1_ref, b1_ref))
    x2 = jax.nn.relu(conv(x1, W2_ref, b2_ref))
    x = x2 + jnp.dot(x1, Wres_ref[...], preferred_element_type=_F32) \
        + bres_ref[...]
    h = jax.nn.relu(jnp.dot(x, Wfc1_ref[...], preferred_element_type=_F32)
                    + bfc1_ref[...])
    A = jnp.dot(h, Wfc2_ref[...], preferred_element_type=_F32) + bfc2_ref[...]

    x_out_ref[...] = x
    A_out_ref[...] = A


def kernel(adj, node_emb, W1, b1, W2, b2, Wres, bres, Wfc1, bfc1, Wfc2, bfc2):
    out = pl.pallas_call(
        _fused,
        out_shape=(
            jax.ShapeDtypeStruct((_N, 128), _F32),
            jax.ShapeDtypeStruct((_N, 1), _F32),
        ),
    )(adj, node_emb,
      W1, b1.reshape(1, -1), W2, b2.reshape(1, -1),
      Wres, bres.reshape(1, -1), Wfc1, bfc1.reshape(1, -1),
      Wfc2, bfc2.reshape(1, -1))
    return out


# speedup vs baseline: 3367.7255x; 1.0064x over previous
import jax
import jax.numpy as jnp
from jax.experimental import pallas as pl

_N = 1024
_F32 = jnp.float32


def _fused(adj_ref, x_ref, W1_ref, b1_ref, W2_ref, b2_ref, Wres_ref,
           bres_ref, Wfc1_ref, bfc1_ref, Wfc2_ref, bfc2_ref,
           x_out_ref, A_out_ref):
    adj = adj_ref[...]
    x0 = x_ref[...]

    adjb = adj.astype(jnp.bfloat16)

    deg = jnp.transpose(jnp.sum(adj, axis=0, keepdims=True)) + 2.0
    dinv = jax.lax.rsqrt(deg)
    dinv2 = 2.0 * dinv * dinv

    def conv(x, W_ref, b_ref):
        xw = jnp.dot(x, W_ref[...], preferred_element_type=_F32)
        t = jax.lax.dot_general(adjb, (dinv * xw).astype(jnp.bfloat16),
                                (((0,), (0,)), ((), ())),
                                preferred_element_type=_F32)
        return dinv * t + dinv2 * xw + b_ref[...]

    def bdot(a, W_ref):
        return jnp.dot(a.astype(jnp.bfloat16), W_ref[...].astype(jnp.bfloat16),
                       preferred_element_type=_F32)

    x1 = jax.nn.relu(conv(x0, W1_ref, b1_ref))
    x2 = jax.nn.relu(conv(x1, W2_ref, b2_ref))
    x = x2 + jnp.dot(x1, Wres_ref[...], preferred_element_type=_F32) \
        + bres_ref[...]
    h = jax.nn.relu(jnp.dot(x, Wfc1_ref[...], preferred_element_type=_F32)
                    + bfc1_ref[...])
    A = jnp.dot(h, Wfc2_ref[...], preferred_element_type=_F32) + bfc2_ref[...]

    x_out_ref[...] = x
    A_out_ref[...] = A


def kernel(adj, node_emb, W1, b1, W2, b2, Wres, bres, Wfc1, bfc1, Wfc2, bfc2):
    out = pl.pallas_call(
        _fused,
        out_shape=(
            jax.ShapeDtypeStruct((_N, 128), _F32),
            jax.ShapeDtypeStruct((_N, 1), _F32),
        ),
    )(adj, node_emb,
      W1, b1.reshape(1, -1), W2, b2.reshape(1, -1),
      Wres, bres.reshape(1, -1), Wfc1, bfc1.reshape(1, -1),
      Wfc2, bfc2.reshape(1, -1))
    return out
